# baseline (speedup 1.0000x reference)
import jax
import jax.numpy as jnp
import numpy as np
from jax.experimental import pallas as pl
from jax.experimental.pallas import tpu as pltpu

B = 16
D = 1024
VOCAB = 100000
NEG = -1e9
BN = 3584
NBLK = (VOCAB + BN - 1) // BN
VPAD = NBLK * BN

_KEY_LO = np.int32(np.int64(-2147483648)
                   - np.int64(np.float32(-3.0e38).view(np.int32))
                   - 1)


def _f32_to_key(x):
    b = jax.lax.bitcast_convert_type(x, jnp.int32)
    return jnp.where(b >= 0, b, jnp.int32(-2147483648) - b - 1)


def _key_to_f32(k):
    b = jnp.where(k >= 0, k, jnp.int32(-2147483648) - k - 1)
    return jax.lax.bitcast_convert_type(b, jnp.float32)


def _fused_kernel(hidden_ref, emb_ref, temp_ref, tp_ref, tk_ref,
                  arg_ref, probs_ref, logp_ref,
                  v_s, m_s, z_s, arg_s):
    j = pl.program_id(0)

    @pl.when(j == 0)
    def _init():
        m_s[...] = jnp.full_like(m_s, -jnp.inf)
        z_s[...] = jnp.zeros_like(z_s)
        arg_s[...] = jnp.zeros_like(arg_s)

    logits = jax.lax.dot_general(
        hidden_ref[...], emb_ref[...], (((1,), (1,)), ((), ())),
        preferred_element_type=jnp.float32)
    logits = logits / temp_ref[...]

    col = j * BN + jax.lax.broadcasted_iota(jnp.int32, (B, BN), 1)
    lw = jnp.where(col < VOCAB, logits, -jnp.inf)
    v_s[:, pl.ds(j * BN, BN)] = lw

    bm = jnp.max(lw, axis=1, keepdims=True)
    barg = jnp.min(jnp.where(lw == bm, col, jnp.int32(2147483647)),
                   axis=1, keepdims=True)

    m_old = m_s[...]
    m_new = jnp.maximum(m_old, bm)
    z_s[...] = (z_s[...] * jnp.exp(m_old - m_new)
                + jnp.sum(jnp.exp(lw - m_new), axis=1, keepdims=True))
    arg_s[...] = jnp.where(bm > m_old, barg, arg_s[...])
    m_s[...] = m_new

    @pl.when(j == NBLK - 1)
    def _select_emit():
        m = m_s[...]
        probs_ref[...] = jnp.exp(v_s[:, :VOCAB] - m)

        budget = tp_ref[...] * z_s[...]
        topk = tk_ref[...]

        lo0 = jnp.full((B, 1), _KEY_LO, jnp.int32)
        hi0 = _f32_to_key(m)

        def body(_, carry):
            lo, hi = carry
            mid = (lo >> 1) + (hi >> 1) + (lo & hi & 1)
            tau = _key_to_f32(mid)
            mask = v_s[:, :VOCAB] > tau
            cnt = jnp.sum(jnp.where(mask, 1.0, 0.0), axis=1, keepdims=True)
            mass = jnp.sum(jnp.where(mask, probs_ref[...], 0.0), axis=1,
                           keepdims=True)
            good = (mass <= budget) & (cnt < topk)
            lo = jnp.where(good, lo, mid)
            hi = jnp.where(good, mid, hi)
            return lo, hi

        lo, _ = jax.lax.fori_loop(0, 32, body, (lo0, hi0))
        t_lo = _key_to_f32(lo)

        v = v_s[:, :VOCAB]
        keep = v > t_lo
        ek = jnp.where(keep, probs_ref[...], 0.0)
        zk = jnp.sum(ek, axis=1, keepdims=True)
        arg_ref[...] = arg_s[...]
        probs_ref[...] = ek * (1.0 / zk)
        logp_ref[...] = (jnp.where(keep, v, NEG) - m) - jnp.log(zk)


@jax.jit
def kernel(hidden_states, embedding, temperatures, top_ps, top_ks):
    temp = temperatures.reshape(B, 1)
    tp = top_ps.reshape(B, 1)
    tk = top_ks.astype(jnp.float32).reshape(B, 1)

    arg, probs, logp = pl.pallas_call(
        _fused_kernel,
        grid=(NBLK,),
        in_specs=[
            pl.BlockSpec((B, D), lambda j: (0, 0)),
            pl.BlockSpec((BN, D), lambda j: (j, 0)),
            pl.BlockSpec((B, 1), lambda j: (0, 0)),
            pl.BlockSpec((B, 1), lambda j: (0, 0)),
            pl.BlockSpec((B, 1), lambda j: (0, 0)),
        ],
        out_specs=[
            pl.BlockSpec((B, 1), lambda j: (0, 0)),
            pl.BlockSpec((B, VOCAB), lambda j: (0, 0)),
            pl.BlockSpec((B, VOCAB), lambda j: (0, 0)),
        ],
        out_shape=[
            jax.ShapeDtypeStruct((B, 1), jnp.int32),
            jax.ShapeDtypeStruct((B, VOCAB), jnp.float32),
            jax.ShapeDtypeStruct((B, VOCAB), jnp.float32),
        ],
        scratch_shapes=[
            pltpu.VMEM((B, VPAD), jnp.float32),
            pltpu.VMEM((B, 1), jnp.float32),
            pltpu.VMEM((B, 1), jnp.float32),
            pltpu.VMEM((B, 1), jnp.int32),
        ],
        compiler_params=pltpu.CompilerParams(
            vmem_limit_bytes=100 * 1024 * 1024),
    )(hidden_states, embedding, temp, tp, tk)

    return arg[:, 0], probs, logp
